# baseline (speedup 1.0000x reference)
import functools

import jax
import jax.numpy as jnp
from jax import lax
from jax.experimental import pallas as pl
from jax.experimental.pallas import tpu as pltpu
from jax.experimental.pallas import tpu_sc as plsc

N_NODES = 10000
T = 8
E = 640000
TT = T * T

NC = 2
NS = 16
NW = NC * NS
C = 128
NCHUNKS = E // C
KPW = NCHUNKS // NW
NREM = NCHUNKS - KPW * NW
NBUF = 6
NQUAD = KPW // NBUF

_mesh = plsc.VectorSubcoreMesh(
    core_axis_name="c", subcore_axis_name="s", num_cores=NC, num_subcores=NS
)


@functools.partial(
    pl.kernel,
    out_type=jax.ShapeDtypeStruct((TT, E), jnp.float32),
    mesh=_mesh,
    compiler_params=pltpu.CompilerParams(needs_layout_passes=False),
    scratch_types=[
        pltpu.VMEM((N_NODES * T,), jnp.float32),
        [pltpu.VMEM((C,), jnp.int32) for _ in range(NBUF)],
        [pltpu.VMEM((C,), jnp.int32) for _ in range(NBUF)],
        [pltpu.VMEM((TT, C), jnp.float32) for _ in range(NBUF)],
        [pltpu.SemaphoreType.DMA for _ in range(NBUF)],
        [pltpu.SemaphoreType.DMA for _ in range(NBUF)],
    ],
)
def _encode(edge_hbm, node_hbm, out_hbm,
            table_v, idx1s, idx2s, outs, isems, osems):
    wid = lax.axis_index("s") * NC + lax.axis_index("c")
    pltpu.sync_copy(node_hbm, table_v)

    def chunk_base(k):
        return (wid + k * NW) * C

    def start_idx(k, bi):
        base = chunk_base(k)
        pltpu.async_copy(edge_hbm.at[pl.ds(base, C)], idx1s[bi], isems[bi])
        pltpu.async_copy(edge_hbm.at[pl.ds(E + base, C)], idx2s[bi], isems[bi])

    def wait_idx(bi):
        pltpu.make_async_copy(edge_hbm.at[pl.ds(0, C)], idx1s[bi], isems[bi]).wait()
        pltpu.make_async_copy(edge_hbm.at[pl.ds(0, C)], idx2s[bi], isems[bi]).wait()

    def wait_out(bi):
        pltpu.make_async_copy(
            outs[bi], out_hbm.at[:, pl.ds(0, C)], osems[bi]
        ).wait()

    def compute_chunk(idx1_v, idx2_v, out_v):
        @plsc.parallel_loop(0, C // 16, 1, unroll=1)
        def group_body(g):
            vs8 = idx1_v[pl.ds(g * 16, 16)] * T
            vd8 = idx2_v[pl.ds(g * 16, 16)] * T
            a = [plsc.load_gather(table_v, [vs8 + i]) for i in range(T)]
            b = [plsc.load_gather(table_v, [vd8 + j]) for j in range(T)]
            for i in range(T):
                for j in range(T):
                    out_v[i * T + j, pl.ds(g * 16, 16)] = a[i] * b[j]

    for bi in range(NBUF):
        start_idx(bi, bi)

    def quad_body(k4, _):
        for bi in range(NBUF):
            k = k4 * NBUF + bi
            wait_idx(bi)

            @pl.when(k4 > 0)
            def _wait_prev():
                wait_out(bi)

            compute_chunk(idx1s[bi], idx2s[bi], outs[bi])
            pltpu.async_copy(
                outs[bi], out_hbm.at[:, pl.ds(chunk_base(k), C)], osems[bi]
            )

            @pl.when(k4 < NQUAD - 1)
            def _prefetch():
                start_idx(k + NBUF, bi)
        return 0

    lax.fori_loop(0, NQUAD, quad_body, 0)
    for bi in range(NBUF):
        wait_out(bi)

    @pl.when(wid < NREM)
    def _leftover():
        base = (KPW * NW + wid) * C
        pltpu.sync_copy(edge_hbm.at[pl.ds(base, C)], idx1s[0])
        pltpu.sync_copy(edge_hbm.at[pl.ds(E + base, C)], idx2s[0])
        compute_chunk(idx1s[0], idx2s[0], outs[0])
        pltpu.sync_copy(outs[0], out_hbm.at[:, pl.ds(base, C)])


def kernel(edge_index, node_type):
    out_t = _encode(edge_index.reshape(-1), node_type.reshape(-1))
    return out_t.T


# speedup vs baseline: 1.0008x; 1.0008x over previous
import functools

import jax
import jax.numpy as jnp
from jax import lax
from jax.experimental import pallas as pl
from jax.experimental.pallas import tpu as pltpu
from jax.experimental.pallas import tpu_sc as plsc

N_NODES = 10000
T = 8
E = 640000
TT = T * T

NC = 2
NS = 16
NW = NC * NS
C = 128
NCHUNKS = E // C
KPW = NCHUNKS // NW
NREM = NCHUNKS - KPW * NW
NBUF = 4
NQUAD = KPW // NBUF

_mesh = plsc.VectorSubcoreMesh(
    core_axis_name="c", subcore_axis_name="s", num_cores=NC, num_subcores=NS
)


@functools.partial(
    pl.kernel,
    out_type=jax.ShapeDtypeStruct((TT, E), jnp.float32),
    mesh=_mesh,
    compiler_params=pltpu.CompilerParams(needs_layout_passes=False),
    scratch_types=[
        pltpu.VMEM((N_NODES * T,), jnp.float32),
        [pltpu.VMEM((C,), jnp.int32) for _ in range(NBUF)],
        [pltpu.VMEM((C,), jnp.int32) for _ in range(NBUF)],
        [pltpu.VMEM((TT, C), jnp.float32) for _ in range(NBUF)],
        [pltpu.SemaphoreType.DMA for _ in range(NBUF)],
        [pltpu.SemaphoreType.DMA for _ in range(NBUF)],
    ],
)
def _encode(edge_hbm, node_hbm, out_hbm,
            table_v, idx1s, idx2s, outs, isems, osems):
    wid = lax.axis_index("s") * NC + lax.axis_index("c")
    pltpu.sync_copy(node_hbm, table_v)

    def chunk_base(k):
        return (wid + k * NW) * C

    def start_idx(k, bi):
        base = chunk_base(k)
        pltpu.async_copy(edge_hbm.at[pl.ds(base, C)], idx1s[bi], isems[bi])
        pltpu.async_copy(edge_hbm.at[pl.ds(E + base, C)], idx2s[bi], isems[bi])

    def wait_idx(bi):
        pltpu.make_async_copy(edge_hbm.at[pl.ds(0, C)], idx1s[bi], isems[bi]).wait()
        pltpu.make_async_copy(edge_hbm.at[pl.ds(0, C)], idx2s[bi], isems[bi]).wait()

    def wait_out(bi):
        pltpu.make_async_copy(
            outs[bi], out_hbm.at[:, pl.ds(0, C)], osems[bi]
        ).wait()

    def compute_chunk(idx1_v, idx2_v, out_v):
        @plsc.parallel_loop(0, C // 16, 1, unroll=1)
        def group_body(g):
            vs8 = idx1_v[pl.ds(g * 16, 16)] * T
            vd8 = idx2_v[pl.ds(g * 16, 16)] * T
            a = [plsc.load_gather(table_v, [vs8 + i]) for i in range(T)]
            b = [plsc.load_gather(table_v, [vd8 + j]) for j in range(T)]
            for i in range(T):
                for j in range(T):
                    out_v[i * T + j, pl.ds(g * 16, 16)] = a[i] * b[j]

    for bi in range(NBUF):
        start_idx(bi, bi)

    def quad_body(k4, _):
        for bi in range(NBUF):
            k = k4 * NBUF + bi
            wait_idx(bi)

            @pl.when(k4 > 0)
            def _wait_prev():
                wait_out(bi)

            compute_chunk(idx1s[bi], idx2s[bi], outs[bi])
            pltpu.async_copy(
                outs[bi], out_hbm.at[:, pl.ds(chunk_base(k), C)], osems[bi]
            )

            @pl.when(k4 < NQUAD - 1)
            def _prefetch():
                start_idx(k + NBUF, bi)
        return 0

    lax.fori_loop(0, NQUAD, quad_body, 0)
    for bi in range(NBUF):
        wait_out(bi)

    @pl.when(wid < NREM)
    def _leftover():
        base = (KPW * NW + wid) * C
        pltpu.sync_copy(edge_hbm.at[pl.ds(base, C)], idx1s[0])
        pltpu.sync_copy(edge_hbm.at[pl.ds(E + base, C)], idx2s[0])
        compute_chunk(idx1s[0], idx2s[0], outs[0])
        pltpu.sync_copy(outs[0], out_hbm.at[:, pl.ds(base, C)])


def kernel(edge_index, node_type):
    out_t = _encode(edge_index.reshape(-1), node_type.reshape(-1))
    return out_t.T
